# baseline (speedup 1.0000x reference)
import jax
import jax.numpy as jnp
from jax.experimental import pallas as pl
from jax.experimental.pallas import tpu as pltpu
from jax.experimental.pallas import tpu_sc as plsc

BATCH = 1024
FEAT = 128
N = 100000
K = 5
CLS = 100

BLK = 2048
NBLK = (N + BLK - 1) // BLK
NSTRIP = BLK // 128
NSEL = K + 1
IMIN = -(1 << 31)
IMAX = (1 << 31) - 1


def _topk_body(f_ref, idx_ref, fm_ref, out_ref, fn_ref, vals_ref, cols_ref):
    i = pl.program_id(0)

    @pl.when(i == 0)
    def _init():
        f = f_ref[...]
        norm = jnp.sqrt(jnp.sum(f * f, axis=1, keepdims=True))
        fn_ref[...] = f / jnp.maximum(norm, 1e-12)
        vals_ref[...] = jnp.full((BATCH, 8), IMIN, jnp.int32)
        cols_ref[...] = jnp.zeros((BATCH, 8), jnp.int32)

    fn = fn_ref[...]
    fm = fm_ref[...]
    dis = jax.lax.dot_general(fn, fm, (((1,), (1,)), ((), ())),
                              preferred_element_type=jnp.float32)
    lim = jnp.where(i == NBLK - 1, N - (NBLK - 1) * BLK, BLK)
    ci = jax.lax.broadcasted_iota(jnp.int32, (BATCH, 128), 1)

    planes = [jnp.full((BATCH, 128), IMIN, jnp.int32) for _ in range(NSEL)]
    for s in range(NSTRIP):
        d = dis[:, s * 128:(s + 1) * 128]
        bits = jax.lax.bitcast_convert_type(d, jnp.int32)
        key = bits ^ (jax.lax.shift_right_arithmetic(bits, 31) & 0x7FFFFFFF)
        x = (key & ~0x7FF) | ((0x7FF - s * 128) - ci)
        x = jnp.where(s * 128 + ci < lim, x, IMIN)
        for k in range(NSEL):
            t = jnp.maximum(planes[k], x)
            if k + 1 < NSEL:
                x = jnp.minimum(planes[k], x)
            planes[k] = t

    run_v = vals_ref[...]
    run_c = cols_ref[...]
    for _ in range(NSEL):
        mm = planes[0]
        for p in planes[1:]:
            mm = jnp.maximum(mm, p)
        m = jnp.max(mm, axis=1, keepdims=True)
        planes = [jnp.where(p == m, IMIN, p) for p in planes]
        colv = i * BLK + (0x7FF - (m & 0x7FF))
        sh_v = jnp.concatenate(
            [jnp.full((BATCH, 1), IMAX, jnp.int32), run_v[:, :7]], axis=1)
        sh_c = jnp.concatenate(
            [jnp.zeros((BATCH, 1), jnp.int32), run_c[:, :7]], axis=1)
        keep = run_v >= m
        above = sh_v >= m
        run_v = jnp.where(keep, run_v, jnp.where(above, m, sh_v))
        run_c = jnp.where(keep, run_c, jnp.where(above, colv, sh_c))
    vals_ref[...] = run_v
    cols_ref[...] = run_c

    @pl.when(i == NBLK - 1)
    def _emit():
        is_self = (run_c == idx_ref[...]).astype(jnp.int32)
        s = is_self
        for sh in (1, 2, 4):
            s = s | jnp.concatenate(
                [jnp.zeros((BATCH, sh), jnp.int32), s[:, :8 - sh]], axis=1)
        nxt_c = jnp.concatenate(
            [run_c[:, 1:], jnp.zeros((BATCH, 1), jnp.int32)], axis=1)
        out_ref[...] = jnp.where(s == 1, nxt_c, run_c)


def _topk_indices(features, idx2d, feat_memory):
    return pl.pallas_call(
        _topk_body,
        grid=(NBLK,),
        in_specs=[
            pl.BlockSpec((BATCH, FEAT), lambda i: (0, 0)),
            pl.BlockSpec((BATCH, 1), lambda i: (0, 0)),
            pl.BlockSpec((BLK, FEAT), lambda i: (i, 0)),
        ],
        out_specs=pl.BlockSpec((BATCH, 8), lambda i: (0, 0)),
        out_shape=jax.ShapeDtypeStruct((BATCH, 8), jnp.int32),
        scratch_shapes=[
            pltpu.VMEM((BATCH, FEAT), jnp.float32),
            pltpu.VMEM((BATCH, 8), jnp.int32),
            pltpu.VMEM((BATCH, 8), jnp.int32),
        ],
        compiler_params=pltpu.CompilerParams(
            dimension_semantics=("arbitrary",)),
    )(features, idx2d, feat_memory)


CPAD = 128
PADBLK = 4000


def _pad_body(p_ref, o_ref):
    o_ref[:, :CLS] = p_ref[...]
    o_ref[:, CLS:] = jnp.zeros((PADBLK, CPAD - CLS), jnp.float32)


def _pad_pred(pred_memory):
    return pl.pallas_call(
        _pad_body,
        grid=(N // PADBLK,),
        in_specs=[pl.BlockSpec((PADBLK, CLS), lambda i: (i, 0))],
        out_specs=pl.BlockSpec((PADBLK, CPAD), lambda i: (i, 0)),
        out_shape=jax.ShapeDtypeStruct((N, CPAD), jnp.float32),
    )(pred_memory)


GW = 128


def _sc_gather(pred_padded, flat_idx):
    mesh = plsc.VectorSubcoreMesh(core_axis_name="c", subcore_axis_name="s")

    @pl.kernel(out_type=jax.ShapeDtypeStruct((BATCH * K, CPAD), jnp.float32),
               mesh=mesh)
    def k(pred_hbm, i_hbm, o_hbm):
        def body(i_vmem, o_vmem):
            pltpu.sync_copy(pred_hbm.at[i_vmem.at[0]], o_vmem)

        pltpu.emit_pipeline(
            body,
            grid=(BATCH * K // GW,),
            in_specs=[pl.BlockSpec((1, GW), lambda i: (0, i))],
            out_specs=[pl.BlockSpec((GW, CPAD), lambda i: (i, 0))],
            core_axis_name=("c", "s"),
            dimension_semantics=(pltpu.PARALLEL,),
        )(i_hbm, o_hbm)

    return k(pred_padded, flat_idx)


def _mean_argmax_body(g_ref, mean_ref, lab_ref):
    acc = g_ref[0:BATCH, :CLS]
    for k in range(1, K):
        acc = acc + g_ref[k * BATCH:(k + 1) * BATCH, :CLS]
    mean = acc * (1.0 / K)
    mean_ref[...] = mean
    m = jnp.max(mean, axis=1, keepdims=True)
    ci = jax.lax.broadcasted_iota(jnp.int32, (BATCH, CLS), 1)
    lab_ref[...] = jnp.min(jnp.where(mean == m, ci, IMAX), axis=1,
                           keepdims=True)


def _mean_argmax(gathered):
    return pl.pallas_call(
        _mean_argmax_body,
        out_shape=[jax.ShapeDtypeStruct((BATCH, CLS), jnp.float32),
                   jax.ShapeDtypeStruct((BATCH, 1), jnp.int32)],
    )(gathered)


def kernel(features, idx, feat_memory, pred_memory):
    idx2d = idx.astype(jnp.int32).reshape(BATCH, 1)
    top = _topk_indices(features, idx2d, feat_memory)
    flat_idx = top[:, :K].T.reshape(1, BATCH * K)
    pred_padded = _pad_pred(pred_memory)
    gathered = _sc_gather(pred_padded, flat_idx)
    mean_logits, lab = _mean_argmax(gathered)
    return lab.reshape(BATCH), mean_logits


# speedup vs baseline: 1.6595x; 1.6595x over previous
import jax
import jax.numpy as jnp
from jax.experimental import pallas as pl
from jax.experimental.pallas import tpu as pltpu
from jax.experimental.pallas import tpu_sc as plsc

BATCH = 1024
FEAT = 128
N = 100000
K = 5
CLS = 100

BLK = 2048
NBLK = (N + BLK - 1) // BLK
NSTRIP = BLK // 128
NSEL = K + 1
IMIN = -(1 << 31)
IMAX = (1 << 31) - 1


def _topk_body(f_ref, idx_ref, fm_ref, out_ref, fn_ref, *plane_refs):
    i = pl.program_id(0)

    @pl.when(i == 0)
    def _init():
        f = f_ref[...]
        norm = jnp.sqrt(jnp.sum(f * f, axis=1, keepdims=True))
        fn_ref[...] = f / jnp.maximum(norm, 1e-12)
        for pr in plane_refs:
            pr[...] = jnp.full((BATCH, 128), IMIN, jnp.int32)

    fn = fn_ref[...]
    fm = fm_ref[...]
    dis = jax.lax.dot_general(fn, fm, (((1,), (1,)), ((), ())),
                              preferred_element_type=jnp.float32)
    lim = jnp.where(i == NBLK - 1, N - (NBLK - 1) * BLK, BLK)
    ci = jax.lax.broadcasted_iota(jnp.int32, (BATCH, 128), 1)

    for s in range(NSTRIP):
        d = dis[:, s * 128:(s + 1) * 128]
        bits = jax.lax.bitcast_convert_type(d, jnp.int32)
        key = bits ^ (jax.lax.shift_right_arithmetic(bits, 31) & 0x7FFFFFFF)
        x = (key & ~0x3FF) | (0x3FF - (i * NSTRIP + s))
        x = jnp.where(s * 128 + ci < lim, x, IMIN)
        for k in range(NSEL):
            pk = plane_refs[k][...]
            t = jnp.maximum(pk, x)
            plane_refs[k][...] = t
            if k + 1 < NSEL:
                x = jnp.minimum(pk, x)

    @pl.when(i == NBLK - 1)
    def _emit():
        planes = [pr[...] for pr in plane_refs]
        cols = []
        for _ in range(NSEL):
            mm = planes[0]
            for p in planes[1:]:
                mm = jnp.maximum(mm, p)
            m = jnp.max(mm, axis=1, keepdims=True)
            lane = jnp.min(jnp.where(mm == m, ci, IMAX), axis=1,
                           keepdims=True)
            chunk = 0x3FF - (m & 0x3FF)
            cols.append(chunk * 128 + lane)
            planes = [jnp.where(p == m, IMIN, p) for p in planes]
        run_c = jnp.concatenate(
            cols + [jnp.zeros((BATCH, 8 - NSEL), jnp.int32)], axis=1)
        s = (run_c == idx_ref[...]).astype(jnp.int32)
        for sh in (1, 2, 4):
            s = s | jnp.concatenate(
                [jnp.zeros((BATCH, sh), jnp.int32), s[:, :8 - sh]], axis=1)
        nxt_c = jnp.concatenate(
            [run_c[:, 1:], jnp.zeros((BATCH, 1), jnp.int32)], axis=1)
        out_ref[...] = jnp.where(s == 1, nxt_c, run_c)


def _topk_indices(features, idx2d, feat_memory):
    return pl.pallas_call(
        _topk_body,
        grid=(NBLK,),
        in_specs=[
            pl.BlockSpec((BATCH, FEAT), lambda i: (0, 0)),
            pl.BlockSpec((BATCH, 1), lambda i: (0, 0)),
            pl.BlockSpec((BLK, FEAT), lambda i: (i, 0)),
        ],
        out_specs=pl.BlockSpec((BATCH, 8), lambda i: (0, 0)),
        out_shape=jax.ShapeDtypeStruct((BATCH, 8), jnp.int32),
        scratch_shapes=[pltpu.VMEM((BATCH, FEAT), jnp.float32)] +
                       [pltpu.VMEM((BATCH, 128), jnp.int32)
                        for _ in range(NSEL)],
        compiler_params=pltpu.CompilerParams(
            dimension_semantics=("arbitrary",)),
    )(features, idx2d, feat_memory)


CPAD = 128
PADBLK = 4000


def _pad_body(p_ref, o_ref):
    o_ref[:, :CLS] = p_ref[...]
    o_ref[:, CLS:] = jnp.zeros((PADBLK, CPAD - CLS), jnp.float32)


def _pad_pred(pred_memory):
    return pl.pallas_call(
        _pad_body,
        grid=(N // PADBLK,),
        in_specs=[pl.BlockSpec((PADBLK, CLS), lambda i: (i, 0))],
        out_specs=pl.BlockSpec((PADBLK, CPAD), lambda i: (i, 0)),
        out_shape=jax.ShapeDtypeStruct((N, CPAD), jnp.float32),
    )(pred_memory)


GW = 128


def _sc_gather(pred_padded, flat_idx):
    mesh = plsc.VectorSubcoreMesh(core_axis_name="c", subcore_axis_name="s")

    @pl.kernel(out_type=jax.ShapeDtypeStruct((BATCH * K, CPAD), jnp.float32),
               mesh=mesh)
    def k(pred_hbm, i_hbm, o_hbm):
        def body(i_vmem, o_vmem):
            pltpu.sync_copy(pred_hbm.at[i_vmem.at[0]], o_vmem)

        pltpu.emit_pipeline(
            body,
            grid=(BATCH * K // GW,),
            in_specs=[pl.BlockSpec((1, GW), lambda i: (0, i))],
            out_specs=[pl.BlockSpec((GW, CPAD), lambda i: (i, 0))],
            core_axis_name=("c", "s"),
            dimension_semantics=(pltpu.PARALLEL,),
        )(i_hbm, o_hbm)

    return k(pred_padded, flat_idx)


def _mean_argmax_body(g_ref, mean_ref, lab_ref):
    acc = g_ref[0:BATCH, :CLS]
    for k in range(1, K):
        acc = acc + g_ref[k * BATCH:(k + 1) * BATCH, :CLS]
    mean = acc * (1.0 / K)
    mean_ref[...] = mean
    m = jnp.max(mean, axis=1, keepdims=True)
    ci = jax.lax.broadcasted_iota(jnp.int32, (BATCH, CLS), 1)
    lab_ref[...] = jnp.min(jnp.where(mean == m, ci, IMAX), axis=1,
                           keepdims=True)


def _mean_argmax(gathered):
    return pl.pallas_call(
        _mean_argmax_body,
        out_shape=[jax.ShapeDtypeStruct((BATCH, CLS), jnp.float32),
                   jax.ShapeDtypeStruct((BATCH, 1), jnp.int32)],
    )(gathered)


def kernel(features, idx, feat_memory, pred_memory):
    idx2d = idx.astype(jnp.int32).reshape(BATCH, 1)
    top = _topk_indices(features, idx2d, feat_memory)
    flat_idx = top[:, :K].T.reshape(1, BATCH * K)
    pred_padded = _pad_pred(pred_memory)
    gathered = _sc_gather(pred_padded, flat_idx)
    mean_logits, lab = _mean_argmax(gathered)
    return lab.reshape(BATCH), mean_logits
